# baseline (speedup 1.0000x reference)
import functools

import jax
import jax.numpy as jnp
from jax import lax
from jax.experimental import pallas as pl
from jax.experimental.pallas import tpu as pltpu
from jax.experimental.pallas import tpu_sc as plsc

VOCAB = 1000000
EMBED = 64
BATCH = 16384
NFIELDS = 3
KOUT = 12
KPAD = 16
HALF = KPAD // 2
VPR = 16

VB = 8192
VGRID = -(-VOCAB // VB)
VPAD = VGRID * VB
QROWS = NFIELDS * VPAD // VPR


def _qproj_body(tT_ref, w1f_ref, w2_ref, b1_ref, b2_ref, q_ref):
    f = pl.program_id(1)
    gf = lax.dot_general(
        w2_ref[...], w1f_ref[0], (((1,), (0,)), ((), ())),
        preferred_element_type=jnp.float32,
    )
    qblk = lax.dot_general(
        tT_ref[...], gf, (((0,), (1,)), ((), ())),
        preferred_element_type=jnp.float32,
    )
    c = lax.dot_general(
        b1_ref[...], w2_ref[...], (((1,), (1,)), ((), ())),
        preferred_element_type=jnp.float32,
    ) + b2_ref[...]
    qblk = qblk + jnp.where(f == 0, 1.0, 0.0) * c
    qblk = jnp.concatenate(
        [qblk, jnp.zeros((VB, KPAD - KOUT), jnp.float32)], axis=1)
    qr = qblk.astype(jnp.bfloat16).astype(jnp.float32)
    b = lax.bitcast_convert_type(qr, jnp.int32)
    packed = jnp.bitwise_or(
        b[:, 0:HALF], lax.shift_right_logical(b[:, HALF:KPAD], 16))
    SUB = VB // VPR
    q_ref[...] = jnp.concatenate(
        [packed[t * SUB:(t + 1) * SUB, :] for t in range(VPR)], axis=1)


_qproj = pl.pallas_call(
    _qproj_body,
    grid=(VGRID, NFIELDS),
    in_specs=[
        pl.BlockSpec((EMBED, VB), lambda i, f: (0, i)),
        pl.BlockSpec((1, EMBED, EMBED), lambda i, f: (f, 0, 0)),
        pl.BlockSpec((KOUT, EMBED), lambda i, f: (0, 0)),
        pl.BlockSpec((1, EMBED), lambda i, f: (0, 0)),
        pl.BlockSpec((1, KOUT), lambda i, f: (0, 0)),
    ],
    out_specs=pl.BlockSpec(
        (VB // VPR, VPR * HALF), lambda i, f: (f * VGRID + i, 0)),
    out_shape=jax.ShapeDtypeStruct((QROWS, VPR * HALF), jnp.int32),
)

ROWS = BATCH * NFIELDS
NC, NS = 2, 16
NW = NC * NS
B_PER_W = ROWS // NW
CHUNK = 128
NCHUNK = B_PER_W // CHUNK
L = 16

_sc_mesh = plsc.VectorSubcoreMesh(core_axis_name="c", subcore_axis_name="s")


@functools.partial(
    pl.kernel,
    mesh=_sc_mesh,
    out_type=jax.ShapeDtypeStruct((ROWS, HALF), jnp.int32),
    scratch_types=[
        pltpu.VMEM((NCHUNK, CHUNK), jnp.int32),
        pltpu.VMEM((CHUNK,), jnp.int32),
        pltpu.VMEM((2, CHUNK, VPR * HALF), jnp.int32),
        pltpu.VMEM((B_PER_W, HALF), jnp.int32),
        pltpu.SemaphoreType.DMA,
    ],
    compiler_params=pltpu.CompilerParams(
        use_tc_tiling_on_sc=False, needs_layout_passes=False),
)
def _gather_sc(idx_hbm, q_hbm, out_hbm, idx_v, row_v, buf_v, sel_v, sem):
    wid = lax.axis_index("s") * NC + lax.axis_index("c")
    pltpu.sync_copy(idx_hbm.at[wid], idx_v)

    lane = lax.iota(jnp.int32, L)

    def start(j):
        for t in range(CHUNK // L):
            w = idx_v[j, pl.ds(t * L, L)]
            row_v[pl.ds(t * L, L)] = jnp.bitwise_or(
                lax.shift_left(lax.shift_right_logical(w, 13), 9),
                w & (VB // VPR - 1))
        return pltpu.async_copy(q_hbm.at[row_v], buf_v.at[j % 2], sem)

    def select(j):
        for t in range(CHUNK // L):
            w = idx_v[j, pl.ds(t * L, L)]
            col0 = (lax.shift_right_logical(w, 9) & (VPR - 1)) * HALF
            rloc = t * L + lane
            orow = j * CHUNK + rloc
            for k in range(HALF):
                vals = plsc.load_gather(buf_v.at[j % 2], [rloc, col0 + k])
                plsc.store_scatter(
                    sel_v, [orow, jnp.full((L,), k, jnp.int32)], vals)

    cp = start(0)
    for j in range(NCHUNK):
        cp.wait()
        if j + 1 < NCHUNK:
            nxt = start(j + 1)
        select(j)
        if j + 1 < NCHUNK:
            cp = nxt
    pltpu.sync_copy(sel_v, out_hbm.at[pl.ds(wid * B_PER_W, B_PER_W)])


_BLK = 4096


def _sum_body(g_ref, o_ref):
    gi = g_ref[...]
    acc = jnp.zeros((_BLK, KOUT), jnp.float32)
    for f in range(NFIELDS):
        gf = gi[:, f * HALF:(f + 1) * HALF]
        hi = lax.bitcast_convert_type(
            jnp.bitwise_and(gf, jnp.int32(-65536)), jnp.float32)
        lo = lax.bitcast_convert_type(
            lax.shift_left(gf, 16), jnp.float32)
        acc = acc + jnp.concatenate([hi, lo[:, :KOUT - HALF]], axis=1)
    o_ref[...] = acc


_fsum = pl.pallas_call(
    _sum_body,
    grid=(BATCH // _BLK,),
    in_specs=[pl.BlockSpec((_BLK, NFIELDS * HALF), lambda i: (i, 0))],
    out_specs=pl.BlockSpec((_BLK, KOUT), lambda i: (i, 0)),
    out_shape=jax.ShapeDtypeStruct((BATCH, KOUT), jnp.float32),
)


def kernel(demo, table, W1, b1, W2, b2):
    tT = table.T
    w1f = W1.reshape(EMBED, NFIELDS, EMBED).transpose(1, 0, 2)
    q = _qproj(tT, w1f, W2, b1.reshape(1, EMBED), b2.reshape(1, KOUT))
    idx = demo + jnp.arange(NFIELDS, dtype=jnp.int32) * VPAD
    idx = idx.reshape(NW, NCHUNK, CHUNK)
    g = _gather_sc(idx, q)
    return _fsum(g.reshape(BATCH, NFIELDS * HALF))


# speedup vs baseline: 2.0138x; 2.0138x over previous
import functools

import jax
import jax.numpy as jnp
from jax import lax
from jax.experimental import pallas as pl
from jax.experimental.pallas import tpu as pltpu
from jax.experimental.pallas import tpu_sc as plsc

VOCAB = 1000000
EMBED = 64
BATCH = 16384
NFIELDS = 3
KOUT = 12
KPAD = 16
VPR = 8

VB = 8192
VGRID = -(-VOCAB // VB)
SUBR = VB // VPR
BROWS = NFIELDS * SUBR
QROWS = VGRID * BROWS


def _qproj_body(tT_ref, w1_ref, w2_ref, b1_ref, b2_ref, q_ref):
    w2 = w2_ref[...]
    gfull = lax.dot_general(
        w2, w1_ref[...], (((1,), (0,)), ((), ())),
        preferred_element_type=jnp.float32,
    )
    zpad = jnp.zeros((KPAD - KOUT, EMBED), jnp.float32)
    g48 = jnp.concatenate(
        [jnp.concatenate([gfull[:, f * EMBED:(f + 1) * EMBED], zpad], axis=0)
         for f in range(NFIELDS)], axis=0)
    qbT = lax.dot_general(
        g48, tT_ref[...], (((1,), (0,)), ((), ())),
        preferred_element_type=jnp.float32,
    )
    c = lax.dot_general(
        b1_ref[...], w2, (((1,), (1,)), ((), ())),
        preferred_element_type=jnp.float32,
    ) + b2_ref[...]
    ccol = jnp.concatenate(
        [jnp.transpose(c),
         jnp.zeros((NFIELDS * KPAD - KOUT, 1), jnp.float32)], axis=0)
    qbT = qbT + ccol
    qb = jnp.transpose(qbT)
    q_ref[...] = jnp.concatenate(
        [jnp.concatenate(
            [qb[t * SUBR:(t + 1) * SUBR, f * KPAD:(f + 1) * KPAD]
             for t in range(VPR)], axis=1)
         for f in range(NFIELDS)], axis=0)


_qproj = pl.pallas_call(
    _qproj_body,
    grid=(VGRID,),
    in_specs=[
        pl.BlockSpec((EMBED, VB), lambda i: (0, i)),
        pl.BlockSpec((EMBED, NFIELDS * EMBED), lambda i: (0, 0)),
        pl.BlockSpec((KOUT, EMBED), lambda i: (0, 0)),
        pl.BlockSpec((1, EMBED), lambda i: (0, 0)),
        pl.BlockSpec((1, KOUT), lambda i: (0, 0)),
    ],
    out_specs=pl.BlockSpec((BROWS, VPR * KPAD), lambda i: (i, 0)),
    out_shape=jax.ShapeDtypeStruct((QROWS, VPR * KPAD), jnp.float32),
    compiler_params=pltpu.CompilerParams(fuse_transposed_lhs_in_matmul=True),
)

ROWS = BATCH * NFIELDS
NC, NS = 2, 16
NW = NC * NS
B_PER_W = ROWS // NW
CHUNK = 128
NCHUNK = B_PER_W // CHUNK
L = 16

_sc_mesh = plsc.VectorSubcoreMesh(core_axis_name="c", subcore_axis_name="s")


@functools.partial(
    pl.kernel,
    mesh=_sc_mesh,
    out_type=jax.ShapeDtypeStruct((ROWS, KOUT), jnp.float32),
    scratch_types=[
        pltpu.VMEM((NCHUNK, CHUNK), jnp.int32),
        pltpu.VMEM((CHUNK,), jnp.int32),
        pltpu.VMEM((2, CHUNK, VPR * KPAD), jnp.float32),
        pltpu.VMEM((B_PER_W, KOUT), jnp.float32),
        pltpu.SemaphoreType.DMA,
    ],
    compiler_params=pltpu.CompilerParams(
        use_tc_tiling_on_sc=False, needs_layout_passes=False),
)
def _gather_sc(idx_hbm, q_hbm, out_hbm, idx_v, row_v, buf_v, sel_v, sem):
    wid = lax.axis_index("s") * NC + lax.axis_index("c")
    pltpu.sync_copy(idx_hbm.at[wid], idx_v)

    lane = lax.iota(jnp.int32, L)

    def start(j):
        for t in range(CHUNK // L):
            w = idx_v[j, pl.ds(t * L, L)]
            row_v[pl.ds(t * L, L)] = lax.shift_right_logical(w, 7)
        return pltpu.async_copy(q_hbm.at[row_v], buf_v.at[j % 2], sem)

    def select(j):
        for t in range(CHUNK // L):
            w = idx_v[j, pl.ds(t * L, L)]
            col0 = w & 127
            rloc = t * L + lane
            orow = j * CHUNK + rloc
            for k in range(KOUT):
                vals = plsc.load_gather(buf_v.at[j % 2], [rloc, col0 + k])
                plsc.store_scatter(
                    sel_v, [orow, jnp.full((L,), k, jnp.int32)], vals)

    cp = start(0)
    for j in range(NCHUNK):
        cp.wait()
        if j + 1 < NCHUNK:
            nxt = start(j + 1)
        select(j)
        if j + 1 < NCHUNK:
            cp = nxt
    pltpu.sync_copy(sel_v, out_hbm.at[pl.ds(wid * B_PER_W, B_PER_W)])


_BLK = 4096


def _sum_body(g_ref, o_ref):
    g = g_ref[...]
    o_ref[...] = (g[:, 0:KOUT] + g[:, KOUT:2 * KOUT]
                  + g[:, 2 * KOUT:3 * KOUT])


_fsum = pl.pallas_call(
    _sum_body,
    grid=(BATCH // _BLK,),
    in_specs=[pl.BlockSpec((_BLK, NFIELDS * KOUT), lambda i: (i, 0))],
    out_specs=pl.BlockSpec((_BLK, KOUT), lambda i: (i, 0)),
    out_shape=jax.ShapeDtypeStruct((BATCH, KOUT), jnp.float32),
)


def kernel(demo, table, W1, b1, W2, b2):
    tT = table.T
    q = _qproj(tT, W1, W2, b1.reshape(1, EMBED), b2.reshape(1, KOUT))
    f_off = jnp.arange(NFIELDS, dtype=jnp.int32) * SUBR
    row = ((demo >> 13) * BROWS + f_off + (demo & (SUBR - 1)))
    col = ((demo >> 10) & (VPR - 1)) * KPAD
    idx = (row * 128 + col).reshape(NW, NCHUNK, CHUNK)
    g = _gather_sc(idx, q)
    return _fsum(g.reshape(BATCH, NFIELDS * KOUT))


# speedup vs baseline: 4.3374x; 2.1539x over previous
import functools

import jax
import jax.numpy as jnp
from jax import lax
from jax.experimental import pallas as pl
from jax.experimental.pallas import tpu as pltpu
from jax.experimental.pallas import tpu_sc as plsc

VOCAB = 1000000
EMBED = 64
BATCH = 16384
NFIELDS = 3
KOUT = 12
KHALF = 6
FROWS = 8
CGROUP = NFIELDS * FROWS

VB = 8192
VGRID = -(-VOCAB // VB)
NCG = VB // 128
BROWS = NCG * CGROUP
QROWS = VGRID * BROWS


def _qproj_body(tT_ref, w1_ref, w2_ref, b1_ref, b2_ref, q_ref):
    w2 = w2_ref[...]
    gfull = lax.dot_general(
        w2, w1_ref[...], (((1,), (0,)), ((), ())),
        preferred_element_type=jnp.float32,
    )
    g36 = jnp.concatenate(
        [gfull[:, f * EMBED:(f + 1) * EMBED] for f in range(NFIELDS)],
        axis=0)
    qbT = lax.dot_general(
        g36, tT_ref[...], (((1,), (0,)), ((), ())),
        preferred_element_type=jnp.float32,
    )
    c = lax.dot_general(
        b1_ref[...], w2, (((1,), (1,)), ((), ())),
        preferred_element_type=jnp.float32,
    ) + b2_ref[...]
    ccol = jnp.concatenate(
        [jnp.transpose(c), jnp.zeros((2 * KOUT, 1), jnp.float32)], axis=0)
    qbT = qbT + ccol
    qr = qbT.astype(jnp.bfloat16).astype(jnp.float32)
    b = lax.bitcast_convert_type(qr, jnp.int32)
    zpad = jnp.zeros((FROWS - KHALF, VB), jnp.int32)
    packs = []
    for f in range(NFIELDS):
        hi = b[f * KOUT:f * KOUT + KHALF, :]
        lo = lax.shift_right_logical(b[f * KOUT + KHALF:(f + 1) * KOUT, :], 16)
        packs.append(jnp.bitwise_or(hi, lo))
        packs.append(zpad)
    p24 = jnp.concatenate(packs, axis=0)
    for cg in range(NCG):
        q_ref[pl.ds(cg * CGROUP, CGROUP), :] = (
            p24[:, cg * 128:(cg + 1) * 128])


_qproj = pl.pallas_call(
    _qproj_body,
    grid=(VGRID,),
    in_specs=[
        pl.BlockSpec((EMBED, VB), lambda i: (0, i)),
        pl.BlockSpec((EMBED, NFIELDS * EMBED), lambda i: (0, 0)),
        pl.BlockSpec((KOUT, EMBED), lambda i: (0, 0)),
        pl.BlockSpec((1, EMBED), lambda i: (0, 0)),
        pl.BlockSpec((1, KOUT), lambda i: (0, 0)),
    ],
    out_specs=pl.BlockSpec((BROWS, 128), lambda i: (i, 0)),
    out_shape=jax.ShapeDtypeStruct((QROWS, 128), jnp.int32),
)

ROWS = BATCH * NFIELDS
NC, NS = 2, 16
NW = NC * NS
B_PER_W = ROWS // NW
CHUNK = 128
NCHUNK = B_PER_W // CHUNK
RING = 16
L = 16

_sc_mesh = plsc.VectorSubcoreMesh(core_axis_name="c", subcore_axis_name="s")


@functools.partial(
    pl.kernel,
    mesh=_sc_mesh,
    out_type=jax.ShapeDtypeStruct((ROWS, L), jnp.int32),
    scratch_types=[
        pltpu.VMEM((B_PER_W,), jnp.int32),
        pltpu.VMEM((RING, KHALF, L), jnp.int32),
        pltpu.VMEM((B_PER_W, L), jnp.int32),
        pltpu.SemaphoreType.DMA,
    ],
    compiler_params=pltpu.CompilerParams(
        use_tc_tiling_on_sc=False, needs_layout_passes=False),
)
def _gather_sc(idx_hbm, q_hbm, out_hbm, ids_v, ring_v, sel_v, sem):
    wid = lax.axis_index("s") * NC + lax.axis_index("c")
    pltpu.sync_copy(idx_hbm.at[wid], ids_v)

    lane = lax.iota(jnp.int32, L)
    rowsel = jnp.minimum(lane, KHALF - 1)

    def fetch(pvecs, i):
        p = pvecs[i // L][i % L]
        r0 = pl.multiple_of(lax.shift_right_logical(p, 7), FROWS)
        l4 = pl.multiple_of(p & 112, L)
        return pltpu.async_copy(
            q_hbm.at[pl.ds(r0, KHALF), pl.ds(l4, L)],
            ring_v.at[i % RING], sem)

    def select(pvecs, c, i):
        l15 = pvecs[i // L] & 15
        vals = plsc.load_gather(
            ring_v.at[i % RING], [rowsel, lane * 0 + l15[i % L]])
        o = c * CHUNK + i
        plsc.store_scatter(sel_v, [lane * 0 + o, lane], vals)

    def chunk_body(c, carry):
        pvecs = [ids_v[pl.ds(c * CHUNK + t * L, L)] for t in range(CHUNK // L)]
        cps = [fetch(pvecs, i) for i in range(RING)]
        for i in range(CHUNK):
            cps[i % RING].wait()
            select(pvecs, c, i)
            if i + RING < CHUNK:
                cps[i % RING] = fetch(pvecs, i + RING)
        return carry

    lax.fori_loop(0, NCHUNK, chunk_body, 0)
    pltpu.sync_copy(sel_v, out_hbm.at[pl.ds(wid * B_PER_W, B_PER_W)])


_BLK = 4096


def _sum_body(g_ref, o_ref):
    gi = g_ref[...]
    acc = jnp.zeros((_BLK, KOUT), jnp.float32)
    for f in range(NFIELDS):
        gf = gi[:, f * L:f * L + KHALF]
        hi = lax.bitcast_convert_type(
            jnp.bitwise_and(gf, jnp.int32(-65536)), jnp.float32)
        lo = lax.bitcast_convert_type(lax.shift_left(gf, 16), jnp.float32)
        acc = acc + jnp.concatenate([hi, lo], axis=1)
    o_ref[...] = acc


_fsum = pl.pallas_call(
    _sum_body,
    grid=(BATCH // _BLK,),
    in_specs=[pl.BlockSpec((_BLK, NFIELDS * L), lambda i: (i, 0))],
    out_specs=pl.BlockSpec((_BLK, KOUT), lambda i: (i, 0)),
    out_shape=jax.ShapeDtypeStruct((BATCH, KOUT), jnp.float32),
)


def kernel(demo, table, W1, b1, W2, b2):
    tT = table.T
    q = _qproj(tT, W1, W2, b1.reshape(1, EMBED), b2.reshape(1, KOUT))
    f_off = jnp.arange(NFIELDS, dtype=jnp.int32) * FROWS
    row = ((demo >> 13) * BROWS + ((demo >> 7) & (NCG - 1)) * CGROUP + f_off)
    idx = (row * 128 + (demo & 127)).reshape(NW, B_PER_W)
    g = _gather_sc(idx, q)
    return _fsum(g.reshape(BATCH, NFIELDS * L))


# speedup vs baseline: 4.4041x; 1.0154x over previous
import functools

import jax
import jax.numpy as jnp
from jax import lax
from jax.experimental import pallas as pl
from jax.experimental.pallas import tpu as pltpu
from jax.experimental.pallas import tpu_sc as plsc

VOCAB = 1000000
EMBED = 64
BATCH = 16384
NFIELDS = 3
KOUT = 12
KHALF = 6
FROWS = 6
CGROUP = NFIELDS * FROWS

VB = 8192
VGRID = -(-VOCAB // VB)
NCG = VB // 128
BROWS = NCG * CGROUP
QROWS = VGRID * BROWS


def _qproj_body(tT_ref, w1_ref, w2_ref, b1_ref, b2_ref, q_ref):
    w2 = w2_ref[...]
    gfull = lax.dot_general(
        w2, w1_ref[...], (((1,), (0,)), ((), ())),
        preferred_element_type=jnp.float32,
    )
    g36 = jnp.concatenate(
        [gfull[:, f * EMBED:(f + 1) * EMBED] for f in range(NFIELDS)],
        axis=0)
    qbT = lax.dot_general(
        g36, tT_ref[...], (((1,), (0,)), ((), ())),
        preferred_element_type=jnp.float32,
    )
    c = lax.dot_general(
        b1_ref[...], w2, (((1,), (1,)), ((), ())),
        preferred_element_type=jnp.float32,
    ) + b2_ref[...]
    ccol = jnp.concatenate(
        [jnp.transpose(c), jnp.zeros((2 * KOUT, 1), jnp.float32)], axis=0)
    qbT = qbT + ccol
    qr = qbT.astype(jnp.bfloat16).astype(jnp.float32)
    b = lax.bitcast_convert_type(qr, jnp.int32)
    packs = []
    for f in range(NFIELDS):
        hi = b[f * KOUT:f * KOUT + KHALF, :]
        lo = lax.shift_right_logical(b[f * KOUT + KHALF:(f + 1) * KOUT, :], 16)
        packs.append(jnp.bitwise_or(hi, lo))
    p24 = jnp.concatenate(packs, axis=0)
    for cg in range(NCG):
        q_ref[pl.ds(cg * CGROUP, CGROUP), :] = (
            p24[:, cg * 128:(cg + 1) * 128])


_qproj = pl.pallas_call(
    _qproj_body,
    grid=(VGRID,),
    in_specs=[
        pl.BlockSpec((EMBED, VB), lambda i: (0, i)),
        pl.BlockSpec((EMBED, NFIELDS * EMBED), lambda i: (0, 0)),
        pl.BlockSpec((KOUT, EMBED), lambda i: (0, 0)),
        pl.BlockSpec((1, EMBED), lambda i: (0, 0)),
        pl.BlockSpec((1, KOUT), lambda i: (0, 0)),
    ],
    out_specs=pl.BlockSpec((BROWS, 128), lambda i: (i, 0)),
    out_shape=jax.ShapeDtypeStruct((QROWS, 128), jnp.int32),
)

ROWS = BATCH * NFIELDS
NC, NS = 2, 16
NW = NC * NS
B_PER_W = ROWS // NW
CHUNK = 128
NCHUNK = B_PER_W // CHUNK
RING = 16
L = 16

_sc_mesh = plsc.VectorSubcoreMesh(core_axis_name="c", subcore_axis_name="s")


@functools.partial(
    pl.kernel,
    mesh=_sc_mesh,
    out_type=jax.ShapeDtypeStruct((ROWS, L), jnp.int32),
    scratch_types=[
        pltpu.VMEM((B_PER_W,), jnp.int32),
        pltpu.VMEM((RING, KHALF, L), jnp.int32),
        pltpu.VMEM((B_PER_W, L), jnp.int32),
        pltpu.SemaphoreType.DMA,
    ],
    compiler_params=pltpu.CompilerParams(
        use_tc_tiling_on_sc=False, needs_layout_passes=False),
)
def _gather_sc(idx_hbm, q_hbm, out_hbm, ids_v, ring_v, sel_v, sem):
    wid = lax.axis_index("s") * NC + lax.axis_index("c")
    pltpu.sync_copy(idx_hbm.at[wid], ids_v)

    lane = lax.iota(jnp.int32, L)
    rowsel = jnp.minimum(lane, KHALF - 1)

    def fetch(pvecs, i):
        p = pvecs[i // L][i % L]
        r0 = lax.shift_right_logical(p, 7)
        l4 = pl.multiple_of(p & 112, L)
        return pltpu.async_copy(
            q_hbm.at[pl.ds(r0, KHALF), pl.ds(l4, L)],
            ring_v.at[i % RING], sem)

    def select(pvecs, c, i):
        l15 = pvecs[i // L] & 15
        vals = plsc.load_gather(
            ring_v.at[i % RING], [rowsel, lane * 0 + l15[i % L]])
        o = c * CHUNK + i
        plsc.store_scatter(sel_v, [lane * 0 + o, lane], vals)

    def chunk_body(c, carry):
        pvecs = [ids_v[pl.ds(c * CHUNK + t * L, L)] for t in range(CHUNK // L)]
        cps = [fetch(pvecs, i) for i in range(RING)]
        for i in range(CHUNK):
            cps[i % RING].wait()
            select(pvecs, c, i)
            if i + RING < CHUNK:
                cps[i % RING] = fetch(pvecs, i + RING)
        return carry

    lax.fori_loop(0, NCHUNK, chunk_body, 0)
    pltpu.sync_copy(sel_v, out_hbm.at[pl.ds(wid * B_PER_W, B_PER_W)])


_BLK = 4096


def _sum_body(g_ref, o_ref):
    gi = g_ref[...]
    acc = jnp.zeros((_BLK, KOUT), jnp.float32)
    for f in range(NFIELDS):
        gf = gi[:, f * L:f * L + KHALF]
        hi = lax.bitcast_convert_type(
            jnp.bitwise_and(gf, jnp.int32(-65536)), jnp.float32)
        lo = lax.bitcast_convert_type(lax.shift_left(gf, 16), jnp.float32)
        acc = acc + jnp.concatenate([hi, lo], axis=1)
    o_ref[...] = acc


_fsum = pl.pallas_call(
    _sum_body,
    grid=(BATCH // _BLK,),
    in_specs=[pl.BlockSpec((_BLK, NFIELDS * L), lambda i: (i, 0))],
    out_specs=pl.BlockSpec((_BLK, KOUT), lambda i: (i, 0)),
    out_shape=jax.ShapeDtypeStruct((BATCH, KOUT), jnp.float32),
)


def kernel(demo, table, W1, b1, W2, b2):
    tT = table.T
    q = _qproj(tT, W1, W2, b1.reshape(1, EMBED), b2.reshape(1, KOUT))
    f_off = jnp.arange(NFIELDS, dtype=jnp.int32) * FROWS
    row = ((demo >> 13) * BROWS + ((demo >> 7) & (NCG - 1)) * CGROUP + f_off)
    idx = (row * 128 + (demo & 127)).reshape(NW, B_PER_W)
    g = _gather_sc(idx, q)
    return _fsum(g.reshape(BATCH, NFIELDS * L))


# speedup vs baseline: 4.9198x; 1.1171x over previous
import functools

import jax
import jax.numpy as jnp
from jax import lax
from jax.experimental import pallas as pl
from jax.experimental.pallas import tpu as pltpu
from jax.experimental.pallas import tpu_sc as plsc

VOCAB = 1000000
EMBED = 64
BATCH = 16384
NFIELDS = 3
KOUT = 12
KHALF = 6
FROWS = 6
CGROUP = NFIELDS * FROWS

VB = 8192
VGRID = -(-VOCAB // VB)
NCG = VB // 128
BROWS = NCG * CGROUP
QROWS = VGRID * BROWS


def _qproj_body(tT_ref, w1_ref, w2_ref, b1_ref, b2_ref, q_ref):
    w2 = w2_ref[...]
    gfull = lax.dot_general(
        w2, w1_ref[...], (((1,), (0,)), ((), ())),
        preferred_element_type=jnp.float32,
    )
    g36 = jnp.concatenate(
        [gfull[:, f * EMBED:(f + 1) * EMBED] for f in range(NFIELDS)],
        axis=0)
    qbT = lax.dot_general(
        g36, tT_ref[...], (((1,), (0,)), ((), ())),
        preferred_element_type=jnp.float32,
    )
    c = lax.dot_general(
        b1_ref[...], w2, (((1,), (1,)), ((), ())),
        preferred_element_type=jnp.float32,
    ) + b2_ref[...]
    ccol = jnp.concatenate(
        [jnp.transpose(c), jnp.zeros((2 * KOUT, 1), jnp.float32)], axis=0)
    qbT = qbT + ccol
    qr = qbT.astype(jnp.bfloat16).astype(jnp.float32)
    b = lax.bitcast_convert_type(qr, jnp.int32)
    packs = []
    for f in range(NFIELDS):
        hi = b[f * KOUT:f * KOUT + KHALF, :]
        lo = lax.shift_right_logical(b[f * KOUT + KHALF:(f + 1) * KOUT, :], 16)
        packs.append(jnp.bitwise_or(hi, lo))
    p24 = jnp.concatenate(packs, axis=0)
    for cg in range(NCG):
        q_ref[pl.ds(cg * CGROUP, CGROUP), :] = (
            p24[:, cg * 128:(cg + 1) * 128])


_qproj = pl.pallas_call(
    _qproj_body,
    grid=(VGRID,),
    in_specs=[
        pl.BlockSpec((EMBED, VB), lambda i: (0, i)),
        pl.BlockSpec((EMBED, NFIELDS * EMBED), lambda i: (0, 0)),
        pl.BlockSpec((KOUT, EMBED), lambda i: (0, 0)),
        pl.BlockSpec((1, EMBED), lambda i: (0, 0)),
        pl.BlockSpec((1, KOUT), lambda i: (0, 0)),
    ],
    out_specs=pl.BlockSpec((BROWS, 128), lambda i: (i, 0)),
    out_shape=jax.ShapeDtypeStruct((QROWS, 128), jnp.int32),
)

ROWS = BATCH * NFIELDS
NC, NS = 2, 16
NW = NC * NS
B_PER_W = ROWS // NW
CHUNK = 128
NCHUNK = B_PER_W // CHUNK
RING = 32
L = 16

_sc_mesh = plsc.VectorSubcoreMesh(core_axis_name="c", subcore_axis_name="s")


@functools.partial(
    pl.kernel,
    mesh=_sc_mesh,
    out_type=jax.ShapeDtypeStruct((ROWS, L), jnp.int32),
    scratch_types=[
        pltpu.VMEM((B_PER_W,), jnp.int32),
        pltpu.VMEM((RING, KHALF, L), jnp.int32),
        pltpu.VMEM((B_PER_W, L), jnp.int32),
        pltpu.SemaphoreType.DMA,
    ],
    compiler_params=pltpu.CompilerParams(
        use_tc_tiling_on_sc=False, needs_layout_passes=False),
)
def _gather_sc(idx_hbm, q_hbm, out_hbm, ids_v, ring_v, sel_v, sem):
    wid = lax.axis_index("s") * NC + lax.axis_index("c")
    pltpu.sync_copy(idx_hbm.at[wid], ids_v)

    lane = lax.iota(jnp.int32, L)
    rowsel = jnp.minimum(lane, KHALF - 1)

    def fetch(pvecs, i):
        p = pvecs[i // L][i % L]
        r0 = lax.shift_right_logical(p, 7)
        l4 = pl.multiple_of(p & 112, L)
        return pltpu.async_copy(
            q_hbm.at[pl.ds(r0, KHALF), pl.ds(l4, L)],
            ring_v.at[i % RING], sem)

    def select(pvecs, c, i):
        l15 = pvecs[i // L] & 15
        vals = plsc.load_gather(
            ring_v.at[i % RING], [rowsel, lane * 0 + l15[i % L]])
        o = c * CHUNK + i
        plsc.store_scatter(sel_v, [lane * 0 + o, lane], vals)

    def chunk_body(c, carry):
        pvecs = [ids_v[pl.ds(c * CHUNK + t * L, L)] for t in range(CHUNK // L)]
        cps = [fetch(pvecs, i) for i in range(RING)]
        for i in range(CHUNK):
            cps[i % RING].wait()
            select(pvecs, c, i)
            if i + RING < CHUNK:
                cps[i % RING] = fetch(pvecs, i + RING)
        return carry

    lax.fori_loop(0, NCHUNK, chunk_body, 0)
    pltpu.sync_copy(sel_v, out_hbm.at[pl.ds(wid * B_PER_W, B_PER_W)])


_BLK = 4096


def _sum_body(g_ref, o_ref):
    gi = g_ref[...]
    acc = jnp.zeros((_BLK, KOUT), jnp.float32)
    for f in range(NFIELDS):
        gf = gi[:, f * L:f * L + KHALF]
        hi = lax.bitcast_convert_type(
            jnp.bitwise_and(gf, jnp.int32(-65536)), jnp.float32)
        lo = lax.bitcast_convert_type(lax.shift_left(gf, 16), jnp.float32)
        acc = acc + jnp.concatenate([hi, lo], axis=1)
    o_ref[...] = acc


_fsum = pl.pallas_call(
    _sum_body,
    grid=(BATCH // _BLK,),
    in_specs=[pl.BlockSpec((_BLK, NFIELDS * L), lambda i: (i, 0))],
    out_specs=pl.BlockSpec((_BLK, KOUT), lambda i: (i, 0)),
    out_shape=jax.ShapeDtypeStruct((BATCH, KOUT), jnp.float32),
)


def kernel(demo, table, W1, b1, W2, b2):
    tT = table.T
    q = _qproj(tT, W1, W2, b1.reshape(1, EMBED), b2.reshape(1, KOUT))
    f_off = jnp.arange(NFIELDS, dtype=jnp.int32) * FROWS
    row = ((demo >> 13) * BROWS + ((demo >> 7) & (NCG - 1)) * CGROUP + f_off)
    idx = (row * 128 + (demo & 127)).reshape(NW, B_PER_W)
    g = _gather_sc(idx, q)
    return _fsum(g.reshape(BATCH, NFIELDS * L))


# speedup vs baseline: 5.2237x; 1.0618x over previous
import functools

import jax
import jax.numpy as jnp
from jax import lax
from jax.experimental import pallas as pl
from jax.experimental.pallas import tpu as pltpu
from jax.experimental.pallas import tpu_sc as plsc

VOCAB = 1000000
EMBED = 64
BATCH = 16384
NFIELDS = 3
KOUT = 12
KHALF = 6
FROWS = 6
CGROUP = NFIELDS * FROWS

VB = 8192
VGRID = -(-VOCAB // VB)
NCG = VB // 128
BROWS = NCG * CGROUP
QROWS = VGRID * BROWS


def _qproj_body(tT_ref, w1_ref, w2_ref, b1_ref, b2_ref, q_ref):
    w2 = w2_ref[...]
    gfull = lax.dot_general(
        w2, w1_ref[...], (((1,), (0,)), ((), ())),
        preferred_element_type=jnp.float32,
    )
    g36 = jnp.concatenate(
        [gfull[:, f * EMBED:(f + 1) * EMBED] for f in range(NFIELDS)],
        axis=0)
    qbT = lax.dot_general(
        g36, tT_ref[...], (((1,), (0,)), ((), ())),
        preferred_element_type=jnp.float32,
    )
    c = lax.dot_general(
        b1_ref[...], w2, (((1,), (1,)), ((), ())),
        preferred_element_type=jnp.float32,
    ) + b2_ref[...]
    ccol = jnp.concatenate(
        [jnp.transpose(c), jnp.zeros((2 * KOUT, 1), jnp.float32)], axis=0)
    qbT = qbT + ccol
    qr = qbT.astype(jnp.bfloat16).astype(jnp.float32)
    b = lax.bitcast_convert_type(qr, jnp.int32)
    packs = []
    for f in range(NFIELDS):
        hi = b[f * KOUT:f * KOUT + KHALF, :]
        lo = lax.shift_right_logical(b[f * KOUT + KHALF:(f + 1) * KOUT, :], 16)
        packs.append(jnp.bitwise_or(hi, lo))
    p24 = jnp.concatenate(packs, axis=0)
    for cg in range(NCG):
        q_ref[pl.ds(cg * CGROUP, CGROUP), :] = (
            p24[:, cg * 128:(cg + 1) * 128])


_qproj = pl.pallas_call(
    _qproj_body,
    grid=(VGRID,),
    in_specs=[
        pl.BlockSpec((EMBED, VB), lambda i: (0, i)),
        pl.BlockSpec((EMBED, NFIELDS * EMBED), lambda i: (0, 0)),
        pl.BlockSpec((KOUT, EMBED), lambda i: (0, 0)),
        pl.BlockSpec((1, EMBED), lambda i: (0, 0)),
        pl.BlockSpec((1, KOUT), lambda i: (0, 0)),
    ],
    out_specs=pl.BlockSpec((BROWS, 128), lambda i: (i, 0)),
    out_shape=jax.ShapeDtypeStruct((QROWS, 128), jnp.int32),
)

ROWS = BATCH * NFIELDS
NC, NS = 2, 16
NW = NC * NS
B_PER_W = ROWS // NW
CHUNK = 128
NCHUNK = B_PER_W // CHUNK
RING = 64
L = 16

_sc_mesh = plsc.VectorSubcoreMesh(core_axis_name="c", subcore_axis_name="s")


@functools.partial(
    pl.kernel,
    mesh=_sc_mesh,
    out_type=jax.ShapeDtypeStruct((ROWS, L), jnp.int32),
    scratch_types=[
        pltpu.VMEM((B_PER_W,), jnp.int32),
        pltpu.VMEM((RING, KHALF, L), jnp.int32),
        pltpu.VMEM((B_PER_W, L), jnp.int32),
        pltpu.SemaphoreType.DMA,
    ],
    compiler_params=pltpu.CompilerParams(
        use_tc_tiling_on_sc=False, needs_layout_passes=False),
)
def _gather_sc(idx_hbm, q_hbm, out_hbm, ids_v, ring_v, sel_v, sem):
    wid = lax.axis_index("s") * NC + lax.axis_index("c")
    pltpu.sync_copy(idx_hbm.at[wid], ids_v)

    lane = lax.iota(jnp.int32, L)
    rowsel = jnp.minimum(lane, KHALF - 1)

    def fetch(pvecs, i):
        p = pvecs[i // L][i % L]
        r0 = lax.shift_right_logical(p, 7)
        l4 = pl.multiple_of(p & 112, L)
        return pltpu.async_copy(
            q_hbm.at[pl.ds(r0, KHALF), pl.ds(l4, L)],
            ring_v.at[i % RING], sem)

    def select(pvecs, c, i):
        l15 = pvecs[i // L] & 15
        vals = plsc.load_gather(
            ring_v.at[i % RING], [rowsel, lane * 0 + l15[i % L]])
        o = c * CHUNK + i
        plsc.store_scatter(sel_v, [lane * 0 + o, lane], vals)

    def chunk_body(c, carry):
        pvecs = [ids_v[pl.ds(c * CHUNK + t * L, L)] for t in range(CHUNK // L)]
        cps = [fetch(pvecs, i) for i in range(RING)]
        for i in range(CHUNK):
            cps[i % RING].wait()
            select(pvecs, c, i)
            if i + RING < CHUNK:
                cps[i % RING] = fetch(pvecs, i + RING)
        return carry

    lax.fori_loop(0, NCHUNK, chunk_body, 0)
    pltpu.sync_copy(sel_v, out_hbm.at[pl.ds(wid * B_PER_W, B_PER_W)])


_BLK = 4096


def _sum_body(g_ref, o_ref):
    gi = g_ref[...]
    acc = jnp.zeros((_BLK, KOUT), jnp.float32)
    for f in range(NFIELDS):
        gf = gi[:, f * L:f * L + KHALF]
        hi = lax.bitcast_convert_type(
            jnp.bitwise_and(gf, jnp.int32(-65536)), jnp.float32)
        lo = lax.bitcast_convert_type(lax.shift_left(gf, 16), jnp.float32)
        acc = acc + jnp.concatenate([hi, lo], axis=1)
    o_ref[...] = acc


_fsum = pl.pallas_call(
    _sum_body,
    grid=(BATCH // _BLK,),
    in_specs=[pl.BlockSpec((_BLK, NFIELDS * L), lambda i: (i, 0))],
    out_specs=pl.BlockSpec((_BLK, KOUT), lambda i: (i, 0)),
    out_shape=jax.ShapeDtypeStruct((BATCH, KOUT), jnp.float32),
)


def kernel(demo, table, W1, b1, W2, b2):
    tT = table.T
    q = _qproj(tT, W1, W2, b1.reshape(1, EMBED), b2.reshape(1, KOUT))
    f_off = jnp.arange(NFIELDS, dtype=jnp.int32) * FROWS
    row = ((demo >> 13) * BROWS + ((demo >> 7) & (NCG - 1)) * CGROUP + f_off)
    idx = (row * 128 + (demo & 127)).reshape(NW, B_PER_W)
    g = _gather_sc(idx, q)
    return _fsum(g.reshape(BATCH, NFIELDS * L))


# speedup vs baseline: 5.2438x; 1.0039x over previous
import functools

import jax
import jax.numpy as jnp
from jax import lax
from jax.experimental import pallas as pl
from jax.experimental.pallas import tpu as pltpu
from jax.experimental.pallas import tpu_sc as plsc

VOCAB = 1000000
EMBED = 64
BATCH = 16384
NFIELDS = 3
KOUT = 12
KHALF = 6
FROWS = 6
CGROUP = NFIELDS * FROWS

VB = 8192
VGRID = -(-VOCAB // VB)
NCG = VB // 128
BROWS = NCG * CGROUP
QROWS = VGRID * BROWS


def _qproj_body(tT_ref, w1_ref, w2_ref, b1_ref, b2_ref, q_ref):
    w2 = w2_ref[...]
    gfull = lax.dot_general(
        w2, w1_ref[...], (((1,), (0,)), ((), ())),
        preferred_element_type=jnp.float32,
    )
    g36 = jnp.concatenate(
        [gfull[:, f * EMBED:(f + 1) * EMBED] for f in range(NFIELDS)],
        axis=0)
    qbT = lax.dot_general(
        g36, tT_ref[...], (((1,), (0,)), ((), ())),
        preferred_element_type=jnp.float32,
    )
    c = lax.dot_general(
        b1_ref[...], w2, (((1,), (1,)), ((), ())),
        preferred_element_type=jnp.float32,
    ) + b2_ref[...]
    ccol = jnp.concatenate(
        [jnp.transpose(c), jnp.zeros((2 * KOUT, 1), jnp.float32)], axis=0)
    qbT = qbT + ccol
    qr = qbT.astype(jnp.bfloat16).astype(jnp.float32)
    b = lax.bitcast_convert_type(qr, jnp.int32)
    packs = []
    for f in range(NFIELDS):
        hi = b[f * KOUT:f * KOUT + KHALF, :]
        lo = lax.shift_right_logical(b[f * KOUT + KHALF:(f + 1) * KOUT, :], 16)
        packs.append(jnp.bitwise_or(hi, lo))
    p24 = jnp.concatenate(packs, axis=0)
    for cg in range(NCG):
        q_ref[pl.ds(cg * CGROUP, CGROUP), :] = (
            p24[:, cg * 128:(cg + 1) * 128])


_qproj = pl.pallas_call(
    _qproj_body,
    grid=(VGRID,),
    in_specs=[
        pl.BlockSpec((EMBED, VB), lambda i: (0, i)),
        pl.BlockSpec((EMBED, NFIELDS * EMBED), lambda i: (0, 0)),
        pl.BlockSpec((KOUT, EMBED), lambda i: (0, 0)),
        pl.BlockSpec((1, EMBED), lambda i: (0, 0)),
        pl.BlockSpec((1, KOUT), lambda i: (0, 0)),
    ],
    out_specs=pl.BlockSpec((BROWS, 128), lambda i: (i, 0)),
    out_shape=jax.ShapeDtypeStruct((QROWS, 128), jnp.int32),
)

ROWS = BATCH * NFIELDS
NC, NS = 2, 16
NW = NC * NS
B_PER_W = ROWS // NW
CHUNK = 128
NCHUNK = B_PER_W // CHUNK
RING = 128
L = 16

_sc_mesh = plsc.VectorSubcoreMesh(core_axis_name="c", subcore_axis_name="s")


@functools.partial(
    pl.kernel,
    mesh=_sc_mesh,
    out_type=jax.ShapeDtypeStruct((ROWS, L), jnp.int32),
    scratch_types=[
        pltpu.VMEM((B_PER_W,), jnp.int32),
        pltpu.VMEM((RING, KHALF, L), jnp.int32),
        pltpu.VMEM((B_PER_W, L), jnp.int32),
        pltpu.SemaphoreType.DMA,
    ],
    compiler_params=pltpu.CompilerParams(
        use_tc_tiling_on_sc=False, needs_layout_passes=False),
)
def _gather_sc(idx_hbm, q_hbm, out_hbm, ids_v, ring_v, sel_v, sem):
    wid = lax.axis_index("s") * NC + lax.axis_index("c")
    pltpu.sync_copy(idx_hbm.at[wid], ids_v)

    lane = lax.iota(jnp.int32, L)
    rowsel = jnp.minimum(lane, KHALF - 1)

    def fetch(pvecs, i):
        p = pvecs[i // L][i % L]
        r0 = lax.shift_right_logical(p, 7)
        l4 = pl.multiple_of(p & 112, L)
        return pltpu.async_copy(
            q_hbm.at[pl.ds(r0, KHALF), pl.ds(l4, L)],
            ring_v.at[i % RING], sem)

    def select(pvecs, c, i):
        l15 = pvecs[i // L] & 15
        vals = plsc.load_gather(
            ring_v.at[i % RING], [rowsel, lane * 0 + l15[i % L]])
        o = c * CHUNK + i
        plsc.store_scatter(sel_v, [lane * 0 + o, lane], vals)

    def chunk_body(c, carry):
        pvecs = [ids_v[pl.ds(c * CHUNK + t * L, L)] for t in range(CHUNK // L)]
        cps = [fetch(pvecs, i) for i in range(RING)]
        for i in range(CHUNK):
            cps[i % RING].wait()
            select(pvecs, c, i)
            if i + RING < CHUNK:
                cps[i % RING] = fetch(pvecs, i + RING)
        return carry

    lax.fori_loop(0, NCHUNK, chunk_body, 0)
    pltpu.sync_copy(sel_v, out_hbm.at[pl.ds(wid * B_PER_W, B_PER_W)])


_BLK = 4096


def _sum_body(g_ref, o_ref):
    gi = g_ref[...]
    acc = jnp.zeros((_BLK, KOUT), jnp.float32)
    for f in range(NFIELDS):
        gf = gi[:, f * L:f * L + KHALF]
        hi = lax.bitcast_convert_type(
            jnp.bitwise_and(gf, jnp.int32(-65536)), jnp.float32)
        lo = lax.bitcast_convert_type(lax.shift_left(gf, 16), jnp.float32)
        acc = acc + jnp.concatenate([hi, lo], axis=1)
    o_ref[...] = acc


_fsum = pl.pallas_call(
    _sum_body,
    grid=(BATCH // _BLK,),
    in_specs=[pl.BlockSpec((_BLK, NFIELDS * L), lambda i: (i, 0))],
    out_specs=pl.BlockSpec((_BLK, KOUT), lambda i: (i, 0)),
    out_shape=jax.ShapeDtypeStruct((BATCH, KOUT), jnp.float32),
)


def kernel(demo, table, W1, b1, W2, b2):
    tT = table.T
    q = _qproj(tT, W1, W2, b1.reshape(1, EMBED), b2.reshape(1, KOUT))
    f_off = jnp.arange(NFIELDS, dtype=jnp.int32) * FROWS
    row = ((demo >> 13) * BROWS + ((demo >> 7) & (NCG - 1)) * CGROUP + f_off)
    idx = (row * 128 + (demo & 127)).reshape(NW, B_PER_W)
    g = _gather_sc(idx, q)
    return _fsum(g.reshape(BATCH, NFIELDS * L))


# speedup vs baseline: 6.3082x; 1.2030x over previous
import functools

import jax
import jax.numpy as jnp
from jax import lax
from jax.experimental import pallas as pl
from jax.experimental.pallas import tpu as pltpu
from jax.experimental.pallas import tpu_sc as plsc

VOCAB = 1000000
EMBED = 64
BATCH = 16384
NFIELDS = 3
KOUT = 12
KHALF = 6
FROWS = 6
CGROUP = NFIELDS * FROWS

VB = 16384
VGRID = -(-VOCAB // VB)
NCG = VB // 128
BROWS = NCG * CGROUP
QROWS = VGRID * BROWS


def _qproj_body(tT_ref, w1_ref, w2_ref, b1_ref, b2_ref, q_ref):
    w2 = w2_ref[...]
    gfull = lax.dot_general(
        w2, w1_ref[...], (((1,), (0,)), ((), ())),
        preferred_element_type=jnp.float32,
    )
    g36 = jnp.concatenate(
        [gfull[:, f * EMBED:(f + 1) * EMBED] for f in range(NFIELDS)],
        axis=0)
    qbT = lax.dot_general(
        g36, tT_ref[...], (((1,), (0,)), ((), ())),
        preferred_element_type=jnp.float32,
    )
    c = lax.dot_general(
        b1_ref[...], w2, (((1,), (1,)), ((), ())),
        preferred_element_type=jnp.float32,
    ) + b2_ref[...]
    ccol = jnp.concatenate(
        [jnp.transpose(c), jnp.zeros((2 * KOUT, 1), jnp.float32)], axis=0)
    qbT = qbT + ccol
    qr = qbT.astype(jnp.bfloat16).astype(jnp.float32)
    b = lax.bitcast_convert_type(qr, jnp.int32)
    packs = []
    for f in range(NFIELDS):
        hi = b[f * KOUT:f * KOUT + KHALF, :]
        lo = lax.shift_right_logical(b[f * KOUT + KHALF:(f + 1) * KOUT, :], 16)
        packs.append(jnp.bitwise_or(hi, lo))
    p24 = jnp.concatenate(packs, axis=0)
    for cg in range(NCG):
        q_ref[pl.ds(cg * CGROUP, CGROUP), :] = (
            p24[:, cg * 128:(cg + 1) * 128])


_qproj = pl.pallas_call(
    _qproj_body,
    grid=(VGRID,),
    in_specs=[
        pl.BlockSpec((EMBED, VB), lambda i: (0, i)),
        pl.BlockSpec((EMBED, NFIELDS * EMBED), lambda i: (0, 0)),
        pl.BlockSpec((KOUT, EMBED), lambda i: (0, 0)),
        pl.BlockSpec((1, EMBED), lambda i: (0, 0)),
        pl.BlockSpec((1, KOUT), lambda i: (0, 0)),
    ],
    out_specs=pl.BlockSpec((BROWS, 128), lambda i: (i, 0)),
    out_shape=jax.ShapeDtypeStruct((QROWS, 128), jnp.int32),
)

ROWS = BATCH * NFIELDS
NC, NS = 2, 16
NW = NC * NS
B_PER_W = ROWS // NW
CHUNK = 128
NCHUNK = B_PER_W // CHUNK
RING = 128
L = 16

_sc_mesh = plsc.VectorSubcoreMesh(core_axis_name="c", subcore_axis_name="s")


@functools.partial(
    pl.kernel,
    mesh=_sc_mesh,
    out_type=jax.ShapeDtypeStruct((ROWS, L), jnp.int32),
    scratch_types=[
        pltpu.VMEM((B_PER_W,), jnp.int32),
        pltpu.VMEM((RING, KHALF, L), jnp.int32),
        pltpu.VMEM((B_PER_W, L), jnp.int32),
        pltpu.SemaphoreType.DMA,
    ],
    compiler_params=pltpu.CompilerParams(
        use_tc_tiling_on_sc=False, needs_layout_passes=False),
)
def _gather_sc(idx_hbm, q_hbm, out_hbm, ids_v, ring_v, sel_v, sem):
    wid = lax.axis_index("s") * NC + lax.axis_index("c")
    pltpu.sync_copy(idx_hbm.at[wid], ids_v)

    lane = lax.iota(jnp.int32, L)
    rowsel = jnp.minimum(lane, KHALF - 1)

    def fetch(pvecs, i):
        p = pvecs[i // L][i % L]
        r0 = lax.shift_right_logical(p, 7)
        l4 = pl.multiple_of(p & 112, L)
        return pltpu.async_copy(
            q_hbm.at[pl.ds(r0, KHALF), pl.ds(l4, L)],
            ring_v.at[i % RING], sem)

    def select(pvecs, c, i):
        l15 = pvecs[i // L] & 15
        vals = plsc.load_gather(
            ring_v.at[i % RING], [rowsel, lane * 0 + l15[i % L]])
        o = c * CHUNK + i
        plsc.store_scatter(sel_v, [lane * 0 + o, lane], vals)

    def chunk_body(c, carry):
        pvecs = [ids_v[pl.ds(c * CHUNK + t * L, L)] for t in range(CHUNK // L)]
        cps = [fetch(pvecs, i) for i in range(RING)]
        for i in range(CHUNK):
            cps[i % RING].wait()
            select(pvecs, c, i)
            if i + RING < CHUNK:
                cps[i % RING] = fetch(pvecs, i + RING)
        return carry

    lax.fori_loop(0, NCHUNK, chunk_body, 0)
    pltpu.sync_copy(sel_v, out_hbm.at[pl.ds(wid * B_PER_W, B_PER_W)])


_BLK = 4096


def _sum_body(g_ref, o_ref):
    gi = g_ref[...]
    acc = jnp.zeros((_BLK, KOUT), jnp.float32)
    for f in range(NFIELDS):
        gf = gi[:, f * L:f * L + KHALF]
        hi = lax.bitcast_convert_type(
            jnp.bitwise_and(gf, jnp.int32(-65536)), jnp.float32)
        lo = lax.bitcast_convert_type(lax.shift_left(gf, 16), jnp.float32)
        acc = acc + jnp.concatenate([hi, lo], axis=1)
    o_ref[...] = acc


_fsum = pl.pallas_call(
    _sum_body,
    grid=(BATCH // _BLK,),
    in_specs=[pl.BlockSpec((_BLK, NFIELDS * L), lambda i: (i, 0))],
    out_specs=pl.BlockSpec((_BLK, KOUT), lambda i: (i, 0)),
    out_shape=jax.ShapeDtypeStruct((BATCH, KOUT), jnp.float32),
)


def kernel(demo, table, W1, b1, W2, b2):
    tT = table.T
    q = _qproj(tT, W1, W2, b1.reshape(1, EMBED), b2.reshape(1, KOUT))
    f_off = jnp.arange(NFIELDS, dtype=jnp.int32) * FROWS
    row = ((demo // VB) * BROWS + ((demo >> 7) & (NCG - 1)) * CGROUP + f_off)
    idx = (row * 128 + (demo & 127)).reshape(NW, B_PER_W)
    g = _gather_sc(idx, q)
    return _fsum(g.reshape(BATCH, NFIELDS * L))


# speedup vs baseline: 6.8552x; 1.0867x over previous
import functools

import jax
import jax.numpy as jnp
from jax import lax
from jax.experimental import pallas as pl
from jax.experimental.pallas import tpu as pltpu
from jax.experimental.pallas import tpu_sc as plsc

VOCAB = 1000000
EMBED = 64
BATCH = 16384
NFIELDS = 3
KOUT = 12
KHALF = 6
FROWS = 6
CGROUP = NFIELDS * FROWS

VB = 32768
VGRID = -(-VOCAB // VB)
NCG = VB // 128
BROWS = NCG * CGROUP
QROWS = VGRID * BROWS


def _qproj_body(tT_ref, w1_ref, w2_ref, b1_ref, b2_ref, q_ref):
    w2 = w2_ref[...]
    gfull = lax.dot_general(
        w2, w1_ref[...], (((1,), (0,)), ((), ())),
        preferred_element_type=jnp.float32,
    )
    g36 = jnp.concatenate(
        [gfull[:, f * EMBED:(f + 1) * EMBED] for f in range(NFIELDS)],
        axis=0)
    qbT = lax.dot_general(
        g36, tT_ref[...], (((1,), (0,)), ((), ())),
        preferred_element_type=jnp.float32,
    )
    c = lax.dot_general(
        b1_ref[...], w2, (((1,), (1,)), ((), ())),
        preferred_element_type=jnp.float32,
    ) + b2_ref[...]
    ccol = jnp.concatenate(
        [jnp.transpose(c), jnp.zeros((2 * KOUT, 1), jnp.float32)], axis=0)
    qbT = qbT + ccol
    qr = qbT.astype(jnp.bfloat16).astype(jnp.float32)
    b = lax.bitcast_convert_type(qr, jnp.int32)
    packs = []
    for f in range(NFIELDS):
        hi = b[f * KOUT:f * KOUT + KHALF, :]
        lo = lax.shift_right_logical(b[f * KOUT + KHALF:(f + 1) * KOUT, :], 16)
        packs.append(jnp.bitwise_or(hi, lo))
    p24 = jnp.concatenate(packs, axis=0)
    for cg in range(NCG):
        q_ref[pl.ds(cg * CGROUP, CGROUP), :] = (
            p24[:, cg * 128:(cg + 1) * 128])


_qproj = pl.pallas_call(
    _qproj_body,
    grid=(VGRID,),
    in_specs=[
        pl.BlockSpec((EMBED, VB), lambda i: (0, i)),
        pl.BlockSpec((EMBED, NFIELDS * EMBED), lambda i: (0, 0)),
        pl.BlockSpec((KOUT, EMBED), lambda i: (0, 0)),
        pl.BlockSpec((1, EMBED), lambda i: (0, 0)),
        pl.BlockSpec((1, KOUT), lambda i: (0, 0)),
    ],
    out_specs=pl.BlockSpec((BROWS, 128), lambda i: (i, 0)),
    out_shape=jax.ShapeDtypeStruct((QROWS, 128), jnp.int32),
)

ROWS = BATCH * NFIELDS
NC, NS = 2, 16
NW = NC * NS
B_PER_W = ROWS // NW
CHUNK = 128
NCHUNK = B_PER_W // CHUNK
RING = 128
L = 16

_sc_mesh = plsc.VectorSubcoreMesh(core_axis_name="c", subcore_axis_name="s")


@functools.partial(
    pl.kernel,
    mesh=_sc_mesh,
    out_type=jax.ShapeDtypeStruct((ROWS, L), jnp.int32),
    scratch_types=[
        pltpu.VMEM((B_PER_W,), jnp.int32),
        pltpu.VMEM((RING, KHALF, L), jnp.int32),
        pltpu.VMEM((B_PER_W, L), jnp.int32),
        pltpu.SemaphoreType.DMA,
    ],
    compiler_params=pltpu.CompilerParams(
        use_tc_tiling_on_sc=False, needs_layout_passes=False),
)
def _gather_sc(idx_hbm, q_hbm, out_hbm, ids_v, ring_v, sel_v, sem):
    wid = lax.axis_index("s") * NC + lax.axis_index("c")
    pltpu.sync_copy(idx_hbm.at[wid], ids_v)

    lane = lax.iota(jnp.int32, L)
    rowsel = jnp.minimum(lane, KHALF - 1)

    def fetch(pvecs, i):
        p = pvecs[i // L][i % L]
        r0 = lax.shift_right_logical(p, 7)
        l4 = pl.multiple_of(p & 112, L)
        return pltpu.async_copy(
            q_hbm.at[pl.ds(r0, KHALF), pl.ds(l4, L)],
            ring_v.at[i % RING], sem)

    def select(pvecs, c, i):
        l15 = pvecs[i // L] & 15
        vals = plsc.load_gather(
            ring_v.at[i % RING], [rowsel, lane * 0 + l15[i % L]])
        o = c * CHUNK + i
        plsc.store_scatter(sel_v, [lane * 0 + o, lane], vals)

    def chunk_body(c, carry):
        pvecs = [ids_v[pl.ds(c * CHUNK + t * L, L)] for t in range(CHUNK // L)]
        cps = [fetch(pvecs, i) for i in range(RING)]
        for i in range(CHUNK):
            cps[i % RING].wait()
            select(pvecs, c, i)
            if i + RING < CHUNK:
                cps[i % RING] = fetch(pvecs, i + RING)
        return carry

    lax.fori_loop(0, NCHUNK, chunk_body, 0)
    pltpu.sync_copy(sel_v, out_hbm.at[pl.ds(wid * B_PER_W, B_PER_W)])


_BLK = 4096


def _sum_body(g_ref, o_ref):
    gi = g_ref[...]
    acc = jnp.zeros((_BLK, KOUT), jnp.float32)
    for f in range(NFIELDS):
        gf = gi[:, f * L:f * L + KHALF]
        hi = lax.bitcast_convert_type(
            jnp.bitwise_and(gf, jnp.int32(-65536)), jnp.float32)
        lo = lax.bitcast_convert_type(lax.shift_left(gf, 16), jnp.float32)
        acc = acc + jnp.concatenate([hi, lo], axis=1)
    o_ref[...] = acc


_fsum = pl.pallas_call(
    _sum_body,
    grid=(BATCH // _BLK,),
    in_specs=[pl.BlockSpec((_BLK, NFIELDS * L), lambda i: (i, 0))],
    out_specs=pl.BlockSpec((_BLK, KOUT), lambda i: (i, 0)),
    out_shape=jax.ShapeDtypeStruct((BATCH, KOUT), jnp.float32),
)


def kernel(demo, table, W1, b1, W2, b2):
    tT = table.T
    q = _qproj(tT, W1, W2, b1.reshape(1, EMBED), b2.reshape(1, KOUT))
    f_off = jnp.arange(NFIELDS, dtype=jnp.int32) * FROWS
    row = ((demo // VB) * BROWS + ((demo >> 7) & (NCG - 1)) * CGROUP + f_off)
    idx = (row * 128 + (demo & 127)).reshape(NW, B_PER_W)
    g = _gather_sc(idx, q)
    return _fsum(g.reshape(BATCH, NFIELDS * L))
